# baseline (speedup 1.0000x reference)
import functools

import jax
import jax.numpy as jnp
from jax import lax
from jax.experimental import pallas as pl
from jax.experimental.pallas import tpu as pltpu
from jax.experimental.pallas import tpu_sc as plsc

_B = 4
_W = 64
_HW = 4096
_NP = 16
_PATCH = 64
_NSUB = 32


def _topk_body(flat_ref, aidx_ref, gum_ref, f_ref, i_ref):
    flat = flat_ref[0]
    gum = gum_ref[0]
    aidx = aidx_ref[0]
    total = jnp.sum(flat)
    probs = flat / total
    pert = jnp.log(probs) + gum
    iota = lax.broadcasted_iota(jnp.int32, (1, _HW), 1)
    col = lax.broadcasted_iota(jnp.int32, (1, 128), 1)
    att_v = jnp.zeros((1, 128), jnp.float32)
    ys_v = jnp.zeros((1, 128), jnp.float32)
    xs_v = jnp.zeros((1, 128), jnp.float32)
    rowi = lax.broadcasted_iota(jnp.int32, (_NP, 16), 0)
    coli = lax.broadcasted_iota(jnp.int32, (_NP, 16), 1)
    pv_t = jnp.zeros((_NP, 16), jnp.int32)
    neg_inf = jnp.float32(-jnp.inf)
    for k in range(_NP):
        m = jnp.max(pert)
        idx = jnp.min(jnp.where(pert == m, iota, _HW))
        hit = iota == idx
        att = jnp.sum(jnp.where(hit, probs, 0.0))
        sel = jnp.sum(jnp.where(hit, aidx, 0))
        pert = jnp.where(hit, neg_inf, pert)
        ys = idx // _W
        xs = idx - ys * _W
        is1 = sel == 1
        y0 = jnp.clip(jnp.where(is1, 8 * ys - 24, 8 * ys - 28), 0, 448)
        x0 = jnp.clip(jnp.where(is1, 8 * xs - 24, 8 * xs - 28), 0, 448)
        ysf = ys.astype(jnp.float32)
        xsf = xs.astype(jnp.float32)
        ys_s = jnp.where(is1, ysf / 2.0, ysf)
        xs_s = jnp.where(is1, xsf / 2.0, xsf)
        ck = col == k
        att_v = jnp.where(ck, att, att_v)
        ys_v = jnp.where(ck, ys_s, ys_v)
        xs_v = jnp.where(ck, xs_s, xs_v)
        onk = rowi == k
        pv_t = jnp.where(onk & (coli == 0), y0, pv_t)
        pv_t = jnp.where(onk & (coli == 1), x0, pv_t)
        pv_t = jnp.where(onk & (coli == 2), sel, pv_t)
    zf = jnp.zeros((1, 128), jnp.float32)
    f_ref[...] = jnp.concatenate([att_v, ys_v, xs_v, zf], axis=0)[None]
    i_ref[...] = pv_t


def _topk_call(flat, aidx, gum):
    return pl.pallas_call(
        _topk_body,
        grid=(_B,),
        in_specs=[
            pl.BlockSpec((1, 1, _HW), lambda b: (b, 0, 0)),
            pl.BlockSpec((1, 1, _HW), lambda b: (b, 0, 0)),
            pl.BlockSpec((1, 1, _HW), lambda b: (b, 0, 0)),
        ],
        out_specs=[
            pl.BlockSpec((1, 4, 128), lambda b: (b, 0, 0)),
            pl.BlockSpec((_NP, 16), lambda b: (b, 0)),
        ],
        out_shape=[
            jax.ShapeDtypeStruct((_B, 4, 128), jnp.float32),
            jax.ShapeDtypeStruct((_B * _NP, 16), jnp.int32),
        ],
    )(flat.reshape(_B, 1, _HW), aidx.reshape(_B, 1, _HW),
      gum.reshape(_B, 1, _HW))


def _gather_body(params_hbm, xh0_hbm, xh1_hbm, out_hbm, pv, buf0, buf1):
    cid = lax.axis_index("c")
    sid = lax.axis_index("s")
    wid = sid * 2 + cid
    pltpu.sync_copy(params_hbm, pv)
    for j, buf in ((0, buf0), (1, buf1)):
        p = wid * 2 + j
        b = p // _NP
        row = pv[p]
        y0 = row[0]
        x0 = row[1]
        sel = row[2]

        def _copy_in(src, b=b, y0=y0, x0=x0, buf=buf):
            pltpu.sync_copy(
                src.at[b, pl.ds(y0, _PATCH), pl.ds(x0, _PATCH), :], buf)

        lax.cond(sel == 1,
                 lambda: _copy_in(xh1_hbm),
                 lambda: _copy_in(xh0_hbm))
        pltpu.sync_copy(buf, out_hbm.at[p])


@functools.cache
def _make_gather_patches():
    return pl.kernel(
        _gather_body,
        out_type=jax.ShapeDtypeStruct((_B * _NP, _PATCH, _PATCH, 3),
                                      jnp.float32),
        mesh=plsc.VectorSubcoreMesh(core_axis_name="c", subcore_axis_name="s"),
        scratch_types=[
            pltpu.VMEM((_B * _NP, 16), jnp.int32),
            pltpu.VMEM((_PATCH, _PATCH, 3), jnp.float32),
            pltpu.VMEM((_PATCH, _PATCH, 3), jnp.float32),
        ],
        compiler_params=pltpu.CompilerParams(use_tc_tiling_on_sc=False),
    )


def _gather_patches(params, xh0, xh1):
    return _make_gather_patches()(params, xh0, xh1)


def kernel(ats_map, ats_index, x_low0, x_low1, x_high0, x_high1, gumbel):
    del x_low0, x_low1
    flat = ats_map.reshape(_B, _HW)
    aidx = ats_index.reshape(_B, _HW)
    f_out, i_out = _topk_call(flat, aidx, gumbel)
    patches = _gather_patches(i_out, x_high0, x_high1)
    patches = patches.reshape(_B, _NP, _PATCH, _PATCH, 3)
    sampled_attention = f_out[:, 0, :_NP]
    samples_out = jnp.stack([f_out[:, 1, :_NP], f_out[:, 2, :_NP]], axis=-1)
    offsets = jnp.zeros_like(samples_out)
    return patches, sampled_attention, offsets, samples_out


# speedup vs baseline: 53.5341x; 53.5341x over previous
import functools

import jax
import jax.numpy as jnp
from jax import lax
from jax.experimental import pallas as pl
from jax.experimental.pallas import tpu as pltpu
from jax.experimental.pallas import tpu_sc as plsc

_B = 4
_W = 64
_HW = 4096
_NP = 16
_PATCH = 64
_NSUB = 32
_WINY = 72
_WINX = 256


def _lane_allreduce(x, op):
    for s in (64, 32, 16, 8, 4, 2, 1):
        x = op(x, pltpu.roll(x, s, 1))
    return x


def _allmax(x):
    return _lane_allreduce(jnp.max(x, axis=0, keepdims=True), jnp.maximum)


def _allmin(x):
    return _lane_allreduce(jnp.min(x, axis=0, keepdims=True), jnp.minimum)


def _allsum(x):
    return _lane_allreduce(jnp.sum(x, axis=0, keepdims=True), jnp.add)


def _topk_body(flat_ref, aidx_ref, gum_ref, f_ref, i_ref):
    iota = (lax.broadcasted_iota(jnp.int32, (32, 128), 0) * 128
            + lax.broadcasted_iota(jnp.int32, (32, 128), 1))
    col = lax.broadcasted_iota(jnp.int32, (1, 128), 1)
    rowi = lax.broadcasted_iota(jnp.int32, (_NP, 16), 0)
    coli = lax.broadcasted_iota(jnp.int32, (_NP, 16), 1)
    neg_inf = jnp.float32(-jnp.inf)
    pert = []
    ord_ = []
    for b in range(_B):
        flat = flat_ref[b]
        pert.append(jnp.log(flat / _allsum(flat)) + gum_ref[b])
        ord_.append(jnp.full((32, 128), -1, jnp.int32))
    for k in range(_NP):
        for b in range(_B):
            m = _allmax(pert[b])
            idx = _allmin(jnp.where(pert[b] == m, iota, _HW))
            hit = iota == idx
            pert[b] = jnp.where(hit, neg_inf, pert[b])
            ord_[b] = jnp.where(hit, k, ord_[b])
    zf = jnp.zeros((1, 128), jnp.float32)
    for b in range(_B):
        flat = flat_ref[b]
        probs = flat / _allsum(flat)
        aidx = aidx_ref[b]
        att_v = jnp.zeros((1, 128), jnp.float32)
        ys_v = jnp.zeros((1, 128), jnp.float32)
        xs_v = jnp.zeros((1, 128), jnp.float32)
        pv_t = jnp.zeros((_NP, 16), jnp.int32)
        for k in range(_NP):
            hit = ord_[b] == k
            att = _allsum(jnp.where(hit, probs, 0.0))
            sel = _allsum(jnp.where(hit, aidx, 0))
            idx = _allmin(jnp.where(hit, iota, _HW))
            ys = idx // _W
            xs = idx - ys * _W
            is1 = sel == 1
            y0 = jnp.clip(jnp.where(is1, 8 * ys - 24, 8 * ys - 28), 0, 448)
            x0 = jnp.clip(jnp.where(is1, 8 * xs - 24, 8 * xs - 28), 0, 448)
            y0a = jnp.minimum(y0 & ~7, 512 - 72)
            dy = y0 - y0a
            xw = jnp.minimum((x0 // 128) * 128, 512 - 256)
            dx = x0 - xw
            ysf = ys.astype(jnp.float32)
            xsf = xs.astype(jnp.float32)
            ys_s = jnp.where(is1, ysf / 2.0, ysf)
            xs_s = jnp.where(is1, xsf / 2.0, xsf)
            ck = col == k
            att_v = jnp.where(ck, att[:, :1], att_v)
            ys_v = jnp.where(ck, ys_s[:, :1], ys_v)
            xs_v = jnp.where(ck, xs_s[:, :1], xs_v)
            onk = rowi == k
            pv_t = jnp.where(onk & (coli == 0), y0a[:, :16], pv_t)
            pv_t = jnp.where(onk & (coli == 1), dy[:, :16], pv_t)
            pv_t = jnp.where(onk & (coli == 2), xw[:, :16], pv_t)
            pv_t = jnp.where(onk & (coli == 3), dx[:, :16], pv_t)
            pv_t = jnp.where(onk & (coli == 4), sel[:, :16], pv_t)
        f_ref[b] = jnp.concatenate([att_v, ys_v, xs_v, zf], axis=0)
        i_ref[pl.ds(_NP * b, _NP), :] = pv_t


def _topk_call(flat, aidx, gum):
    return pl.pallas_call(
        _topk_body,
        out_shape=[
            jax.ShapeDtypeStruct((_B, 4, 128), jnp.float32),
            jax.ShapeDtypeStruct((_B * _NP, 16), jnp.int32),
        ],
    )(flat.reshape(_B, 32, 128), aidx.reshape(_B, 32, 128),
      gum.reshape(_B, 32, 128))


def _gather_body(params_hbm, xh0_hbm, xh1_hbm, out_hbm, pv, win, stg):
    cid = lax.axis_index("c")
    sid = lax.axis_index("s")
    wid = sid * 2 + cid
    pltpu.sync_copy(params_hbm, pv)
    lane = lax.iota(jnp.int32, 16)
    for j in range(2):
        p = wid * 2 + j
        b = p // _NP
        row = pv[p]
        y0a = pl.multiple_of(row[0], 8)
        dy = row[1]
        xw = pl.multiple_of(row[2], 128)
        dx = row[3]
        sel = row[4]

        def _copy_in(src, b=b, y0a=y0a, xw=xw):
            pltpu.sync_copy(
                src.at[b, :, pl.ds(y0a, _WINY), pl.ds(xw, _WINX)], win)

        lax.cond(sel == 1,
                 lambda: _copy_in(xh1_hbm),
                 lambda: _copy_in(xh0_hbm))

        def _shift_row(i, _, dy=dy, dx=dx):
            c = i // _PATCH
            yy = i - c * _PATCH
            ci = jnp.full((16,), c, jnp.int32)
            yi = jnp.full((16,), dy + yy, jnp.int32)
            for xi in range(4):
                vals = plsc.load_gather(win, [ci, yi, dx + 16 * xi + lane])
                stg[c, yy, pl.ds(16 * xi, 16)] = vals
            return 0

        lax.fori_loop(0, 3 * _PATCH, _shift_row, 0)
        pltpu.sync_copy(stg, out_hbm.at[p])


@functools.cache
def _make_gather_patches():
    return pl.kernel(
        _gather_body,
        out_type=jax.ShapeDtypeStruct((_B * _NP, 3, _PATCH, _PATCH),
                                      jnp.float32),
        mesh=plsc.VectorSubcoreMesh(core_axis_name="c", subcore_axis_name="s"),
        scratch_types=[
            pltpu.VMEM((_B * _NP, 16), jnp.int32),
            pltpu.VMEM((3, _WINY, _WINX), jnp.float32),
            pltpu.VMEM((3, _PATCH, _PATCH), jnp.float32),
        ],
        compiler_params=pltpu.CompilerParams(needs_layout_passes=False),
    )


def _gather_patches(params, xh0, xh1):
    return _make_gather_patches()(params, xh0, xh1)


def kernel(ats_map, ats_index, x_low0, x_low1, x_high0, x_high1, gumbel):
    del x_low0, x_low1
    flat = ats_map.reshape(_B, _HW)
    aidx = ats_index.reshape(_B, _HW)
    f_out, i_out = _topk_call(flat, aidx, gumbel)
    xh0p = x_high0.transpose(0, 3, 1, 2)
    xh1p = x_high1.transpose(0, 3, 1, 2)
    patches = _gather_patches(i_out, xh0p, xh1p)
    patches = patches.reshape(_B, _NP, 3, _PATCH, _PATCH)
    patches = patches.transpose(0, 1, 3, 4, 2)
    sampled_attention = f_out[:, 0, :_NP]
    samples_out = jnp.stack([f_out[:, 1, :_NP], f_out[:, 2, :_NP]], axis=-1)
    offsets = jnp.zeros_like(samples_out)
    return patches, sampled_attention, offsets, samples_out
